# baseline (speedup 1.0000x reference)
import functools

import jax
import jax.numpy as jnp
from jax import lax
from jax.experimental import pallas as pl
from jax.experimental.pallas import tpu as pltpu
from jax.experimental.pallas import tpu_sc as plsc

_INFO = plsc.get_sparse_core_info()
_NC = _INFO.num_cores
_NS = _INFO.num_subcores
_NW = _NC * _NS
_CHUNK = 128


def _sc_gather_body(uid_hbm, iid_hbm, ug_t, ig_t, um_t, im_t,
                    out_ug, out_ig, out_um, out_im,
                    uidx_v, iidx_v, r_ug, r_ig, r_um, r_im, sem,
                    *, rows_per_w, bpw):
    wid = lax.axis_index("s") * _NC + lax.axis_index("c")
    rbase = wid * rows_per_w
    base = wid * bpw
    pltpu.sync_copy(uid_hbm.at[pl.ds(rbase, rows_per_w)], uidx_v)
    pltpu.sync_copy(iid_hbm.at[pl.ds(rbase, rows_per_w)], iidx_v)
    handles = []
    for j in range(rows_per_w):
        sl = pl.ds(j * _CHUNK, _CHUNK)
        handles.append(pltpu.async_copy(ug_t.at[uidx_v.at[j]], r_ug.at[sl], sem))
        handles.append(pltpu.async_copy(ig_t.at[iidx_v.at[j]], r_ig.at[sl], sem))
        handles.append(pltpu.async_copy(um_t.at[uidx_v.at[j]], r_um.at[sl], sem))
        handles.append(pltpu.async_copy(im_t.at[iidx_v.at[j]], r_im.at[sl], sem))
    for h in handles:
        h.wait()
    pltpu.sync_copy(r_ug, out_ug.at[pl.ds(base, bpw)])
    pltpu.sync_copy(r_ig, out_ig.at[pl.ds(base, bpw)])
    pltpu.sync_copy(r_um, out_um.at[pl.ds(base, bpw)])
    pltpu.sync_copy(r_im, out_im.at[pl.ds(base, bpw)])


@functools.partial(jax.jit, static_argnames=("b", "d"))
def _sc_gather(uid2d, iid2d, ug_t, ig_t, um_t, im_t, *, b, d):
    bpw = b // _NW
    rows_per_w = bpw // _CHUNK
    mesh = plsc.VectorSubcoreMesh(core_axis_name="c", subcore_axis_name="s")
    row = jax.ShapeDtypeStruct((b, d), jnp.float32)
    f = pl.kernel(
        functools.partial(_sc_gather_body, rows_per_w=rows_per_w, bpw=bpw),
        mesh=mesh,
        out_type=(row, row, row, row),
        compiler_params=pltpu.CompilerParams(use_tc_tiling_on_sc=False),
        scratch_types=[
            pltpu.VMEM((rows_per_w, _CHUNK), jnp.int32),
            pltpu.VMEM((rows_per_w, _CHUNK), jnp.int32),
            pltpu.VMEM((bpw, d), jnp.float32),
            pltpu.VMEM((bpw, d), jnp.float32),
            pltpu.VMEM((bpw, d), jnp.float32),
            pltpu.VMEM((bpw, d), jnp.float32),
            pltpu.SemaphoreType.DMA,
        ],
    )
    return f(uid2d, iid2d, ug_t, ig_t, um_t, im_t)


def _tc_mlp_body(ug, ig, um, im, w1a, w1b, b1, w2, b2, wfa, wfb, bf, out):
    h = um[...] @ w1a[...] + im[...] @ w1b[...] + b1[...]
    h = jnp.maximum(h, 0.0)
    h = jnp.maximum(h @ w2[...] + b2[...], 0.0)
    mf = ug[...] * ig[...]
    out[...] = mf @ wfa[...] + h @ wfb[...] + bf[...]


@functools.partial(jax.jit, static_argnames=("blk",))
def _tc_mlp(ug, ig, um, im, w1a, w1b, b1, w2, b2, wfa, wfb, bf, *, blk):
    b, d = ug.shape
    grid = (b // blk,)
    emb_spec = pl.BlockSpec((blk, d), lambda i: (i, 0))
    full = lambda a: pl.BlockSpec(a.shape, lambda i: (0,) * a.ndim)
    return pl.pallas_call(
        _tc_mlp_body,
        grid=grid,
        in_specs=[emb_spec, emb_spec, emb_spec, emb_spec,
                  full(w1a), full(w1b), full(b1), full(w2), full(b2),
                  full(wfa), full(wfb), full(bf)],
        out_specs=pl.BlockSpec((blk, 1), lambda i: (i, 0)),
        out_shape=jax.ShapeDtypeStruct((b, 1), jnp.float32),
    )(ug, ig, um, im, w1a, w1b, b1, w2, b2, wfa, wfb, bf)


def kernel(inputs, user_emb_gmf, item_emb_gmf, user_emb_mlp, item_emb_mlp,
           W1, b1, W2, b2, Wf, bf):
    b = inputs.shape[0]
    d = user_emb_gmf.shape[1]
    uid2d = inputs[:, 0].reshape(b // _CHUNK, _CHUNK)
    iid2d = inputs[:, 1].reshape(b // _CHUNK, _CHUNK)
    ug, ig, um, im = _sc_gather(uid2d, iid2d, user_emb_gmf, item_emb_gmf,
                                user_emb_mlp, item_emb_mlp, b=b, d=d)
    out = _tc_mlp(ug, ig, um, im,
                  W1[:d], W1[d:], b1.reshape(1, -1),
                  W2, b2.reshape(1, -1),
                  Wf[:d], Wf[d:], bf.reshape(1, 1),
                  blk=2048)
    return out
